# baseline (speedup 1.0000x reference)
import functools

import jax
import jax.numpy as jnp
from jax import lax
from jax.experimental import pallas as pl
from jax.experimental.pallas import tpu as pltpu
from jax.experimental.pallas import tpu_sc as plsc

_NL = 100000
_EL = 3200000
_EG = 1600000
_NSEG = 1022
_SEQ = 1024
_NPAD = 100352
_NT = 196
_ROWS_L = _EL // 128
_ROWS_G = _EG // 128
_RPT = _NPAD // 16


def _sc_body(w_l, k_l, d_l, p_l, w_g, k_g, d_g, p_g, zz,
             out_l, out_g,
             accl, accg, psl, psg, wbuf, kbuf, dbuf, msg, sem_in, sem_sc):
  c = lax.axis_index("c")
  s = lax.axis_index("s")
  w = s * 2 + c
  iota16 = lax.iota(jnp.int32, 16)

  pltpu.sync_copy(zz, msg)
  base = s * _RPT
  for t in range(6):
    pltpu.sync_copy(msg, accl.at[pl.ds(base + t * 1024, 1024)])
    pltpu.sync_copy(msg, accg.at[pl.ds(base + t * 1024, 1024)])
  pltpu.sync_copy(msg.at[pl.ds(0, 128)], accl.at[pl.ds(base + 6144, 128)])
  pltpu.sync_copy(msg.at[pl.ds(0, 128)], accg.at[pl.ds(base + 6144, 128)])
  pltpu.sync_copy(p_l, psl)
  pltpu.sync_copy(p_g, psg)
  plsc.subcore_barrier()

  def run_graph(w_hbm, k_hbm, d_hbm, ps, acc, full_chunks, extra_pred,
                tail_base, tail_workers):
    lw = [ps[j] for j in range(25)]
    lb = [ps[25 + j] for j in range(5)]

    def do_rows(row0, nrows):
      e0 = row0 * 128
      cps = [
          pltpu.async_copy(w_hbm.at[pl.ds(e0, nrows * 128)],
                           wbuf.at[pl.ds(0, nrows * 128)], sem_in),
          pltpu.async_copy(k_hbm.at[pl.ds(e0, nrows * 128)],
                           kbuf.at[pl.ds(0, nrows * 128)], sem_in),
      ] + [
          pltpu.async_copy(d_hbm.at[1, pl.ds(e0 + jr * 128, 128)],
                           dbuf.at[jr], sem_in)
          for jr in range(nrows)
      ]
      for cp in cps:
        cp.wait()

      def grp(g, carry):
        wv = wbuf[pl.ds(g * 16, 16)]
        rows = g * 16 + iota16
        kv0 = plsc.load_gather(kbuf, [rows, jnp.full((16,), 0, jnp.int32)])
        kv1 = plsc.load_gather(kbuf, [rows, jnp.full((16,), 1, jnp.int32)])
        kv2 = plsc.load_gather(kbuf, [rows, jnp.full((16,), 2, jnp.int32)])
        kv3 = plsc.load_gather(kbuf, [rows, jnp.full((16,), 3, jnp.int32)])
        for j in range(5):
          a = (lb[j] + lw[5 * j] * wv + lw[5 * j + 1] * kv0
               + lw[5 * j + 2] * kv1 + lw[5 * j + 3] * kv2
               + lw[5 * j + 4] * kv3)
          plsc.store_scatter(msg, [rows, jnp.full((16,), j, jnp.int32)],
                             jnp.maximum(a, 0.0))
        return carry

      lax.fori_loop(0, nrows * 8, grp, 0)
      hs = [
          pltpu.async_copy(msg.at[pl.ds(jr * 128, 128)],
                           acc.at[dbuf.at[jr]], sem_sc, add=True)
          for jr in range(nrows)
      ]
      for h in hs:
        h.wait()

    def chunk(i, carry):
      do_rows((w + 32 * i) * 8, 8)
      return carry

    lax.fori_loop(0, full_chunks, chunk, 0)

    @pl.when(w < tail_workers)
    def _():
      do_rows(tail_base + w, 1)

  run_graph(w_l, k_l, d_l, psl, accl,
            97 + jnp.where(w < 21, 1, 0).astype(jnp.int32), None, 25000, 0)
  run_graph(w_g, k_g, d_g, psg, accg,
            48 + jnp.where(w < 26, 1, 0).astype(jnp.int32), None, 12496, 4)

  plsc.subcore_barrier()
  pltpu.sync_copy(accl.at[pl.ds(base, _RPT)], out_l.at[c, pl.ds(base, _RPT)])
  pltpu.sync_copy(accg.at[pl.ds(base, _RPT)], out_g.at[c, pl.ds(base, _RPT)])


_sc_edges = functools.partial(
    pl.kernel,
    out_type=[
        jax.ShapeDtypeStruct((2, _NPAD, 8), jnp.float32),
        jax.ShapeDtypeStruct((2, _NPAD, 8), jnp.float32),
    ],
    mesh=plsc.VectorSubcoreMesh(core_axis_name="c", subcore_axis_name="s"),
    compiler_params=pltpu.CompilerParams(needs_layout_passes=False,
                                         use_tc_tiling_on_sc=False),
    scratch_types=[
        pltpu.VMEM_SHARED((_NPAD, 8), jnp.float32),
        pltpu.VMEM_SHARED((_NPAD, 8), jnp.float32),
        pltpu.VMEM((32, 16), jnp.float32),
        pltpu.VMEM((32, 16), jnp.float32),
        pltpu.VMEM((1024,), jnp.float32),
        pltpu.VMEM((1024, 4), jnp.float32),
        pltpu.VMEM((8, 128), jnp.int32),
        pltpu.VMEM((1024, 8), jnp.float32),
        pltpu.SemaphoreType.DMA,
        pltpu.SemaphoreType.DMA,
    ],
)(_sc_body)


def _tc_body(al_ref, ag_ref, bt_ref, w1l, b1l, w2l, b2l, w1g, b1g, w2g, b2g,
             wv, bv, wo, bo, wa, ba, pp, pb, out_ref, sums, cnts, gsum):
  i = pl.program_id(0)

  @pl.when(i == 0)
  def _():
    sums[...] = jnp.zeros((1024, 128), jnp.float32)
    cnts[...] = jnp.zeros((1024, 8), jnp.float32)
    gsum[...] = jnp.zeros((1, 128), jnp.float32)

  @pl.when(i < _NT)
  def _():
    al = al_ref[0] + al_ref[1] + 1.0
    ul = jnp.maximum(
        jnp.dot(al, w1l[...], preferred_element_type=jnp.float32) + b1l[...],
        0.0)
    ml = jnp.dot(ul, w2l[...], preferred_element_type=jnp.float32) + b2l[...]
    xl = jnp.where(ml > 0, ml, 0.01 * ml)
    bt = bt_ref[0]
    seg = lax.broadcasted_iota(jnp.int32, (1024, 512), 0)
    oh = (seg == jnp.broadcast_to(bt, (1024, 512))).astype(jnp.bfloat16)
    sums[...] += jnp.dot(oh, xl.astype(jnp.bfloat16),
                         preferred_element_type=jnp.float32)
    cnts[...] += jnp.dot(oh, jnp.ones((512, 8), jnp.bfloat16),
                         preferred_element_type=jnp.float32)

    ag = ag_ref[0] + ag_ref[1] + 1.0
    ug = jnp.maximum(
        jnp.dot(ag, w1g[...], preferred_element_type=jnp.float32) + b1g[...],
        0.0)
    mg = jnp.dot(ug, w2g[...], preferred_element_type=jnp.float32) + b2g[...]
    xg = jnp.where(mg > 0, mg, 0.01 * mg)
    rid = lax.broadcasted_iota(jnp.int32, (512, 128), 0) + i * 512
    xg = jnp.where(rid < _NL, xg, 0.0)
    gsum[...] += jnp.sum(xg, axis=0, keepdims=True)

  @pl.when(i == _NT)
  def _():
    cnt = jnp.maximum(cnts[...][:, 0:1], 1.0)
    emb = sums[...] / cnt
    gm = gsum[...] * (1.0 / _NL)
    v1 = jnp.dot(gm, wv[...], preferred_element_type=jnp.float32) + bv[...]
    v2 = jnp.dot(v1, wo[...], preferred_element_type=jnp.float32) + bo[...]
    cv = jnp.dot(v2, wa[...], preferred_element_type=jnp.float32) + ba[...]
    fused = emb + cv
    out_ref[...] = jnp.dot(pp[...], fused,
                           preferred_element_type=jnp.float32) + pb[...]


def _const2(i):
  return (0, 0)


def _tc_finish(aggr_l, aggr_g, batch3d, args):
  specs = [
      pl.BlockSpec((2, 512, 8), lambda i: (0, jnp.minimum(i, _NT - 1), 0)),
      pl.BlockSpec((2, 512, 8), lambda i: (0, jnp.minimum(i, _NT - 1), 0)),
      pl.BlockSpec((1, 1, 512), lambda i: (jnp.minimum(i, _NT - 1), 0, 0)),
  ] + [pl.BlockSpec(a.shape, _const2) for a in args]
  return pl.pallas_call(
      _tc_body,
      grid=(_NT + 1,),
      in_specs=specs,
      out_specs=pl.BlockSpec((1024, 128), _const2),
      out_shape=jax.ShapeDtypeStruct((1024, 128), jnp.float32),
      scratch_shapes=[
          pltpu.VMEM((1024, 128), jnp.float32),
          pltpu.VMEM((1024, 8), jnp.float32),
          pltpu.VMEM((1, 128), jnp.float32),
      ],
  )(aggr_l, aggr_g, batch3d, *args)


def kernel(local_edge_index, local_weight, local_kmer, local_batch,
           global_edge_index, global_weight, global_kmer,
           lin_local_w, lin_local_b, mlp_local_w1, mlp_local_b1,
           mlp_local_w2, mlp_local_b2, lin_global_w, lin_global_b,
           mlp_global_w1, mlp_global_b1, mlp_global_w2, mlp_global_b2,
           in_proj_w, in_proj_b, out_proj_w, out_proj_b,
           attn_proj_w, attn_proj_b, project_w, project_b):
  f32 = jnp.float32
  dl = local_edge_index.astype(jnp.int32)
  dg = global_edge_index.astype(jnp.int32)
  kl = local_kmer
  kg = global_kmer

  def mk_params(lw, lb):
    m = jnp.concatenate([lw.reshape(25), 1.0 + lb, jnp.zeros((2,), f32)])
    return jnp.tile(m[:, None], (1, 16)).astype(f32)

  aggr_l, aggr_g = _sc_edges(
      local_weight, kl, dl, mk_params(lin_local_w, lin_local_b),
      global_weight, kg, dg, mk_params(lin_global_w, lin_global_b),
      jnp.zeros((1024, 8), f32))

  batch3d = jnp.pad(local_batch.astype(jnp.int32), (0, _NPAD - _NL),
                    constant_values=1023).reshape(_NT, 1, 512)

  def w1pad(w1):
    return jnp.zeros((8, 128), f32).at[:5].set(w1.T)

  args = [
      w1pad(mlp_local_w1), mlp_local_b1[None], mlp_local_w2.T,
      mlp_local_b2[None],
      w1pad(mlp_global_w1), mlp_global_b1[None], mlp_global_w2.T,
      mlp_global_b2[None],
      in_proj_w[256:384].T, in_proj_b[256:384][None],
      out_proj_w.T, out_proj_b[None],
      attn_proj_w.T, attn_proj_b[None],
      jnp.zeros((1024, 1024), f32).at[:, :_NSEG].set(project_w),
      jnp.broadcast_to(project_b[:, None], (1024, 128)),
  ]
  out1 = _tc_finish(aggr_l, aggr_g, batch3d, args)
  return (out1.reshape(1, _SEQ, 128), jnp.ones((_NSEG, 1, 1), f32))


# speedup vs baseline: 8.8960x; 8.8960x over previous
import functools

import jax
import jax.numpy as jnp
from jax import lax
from jax.experimental import pallas as pl
from jax.experimental.pallas import tpu as pltpu
from jax.experimental.pallas import tpu_sc as plsc

_NL = 100000
_EL = 3200000
_EG = 1600000
_NSEG = 1022
_SEQ = 1024
_NPAD = 100352
_NT = 196
_ROWS_L = _EL // 128
_ROWS_G = _EG // 128
_RPT = _NPAD // 16


def _sc_body(w_l, k_l, d_l, p_l, w_g, k_g, d_g, p_g, zz,
             out_l, out_g,
             accl, accg, psl, psg, wbuf, kbuf, dbuf, msg, sem_in, sem_sc):
  c = lax.axis_index("c")
  s = lax.axis_index("s")
  w = s * 2 + c
  iota16 = lax.iota(jnp.int32, 16)

  pltpu.sync_copy(zz, msg)
  base = s * _RPT
  for t in range(6):
    pltpu.sync_copy(msg, accl.at[pl.ds(base + t * 1024, 1024)])
    pltpu.sync_copy(msg, accg.at[pl.ds(base + t * 1024, 1024)])
  pltpu.sync_copy(msg.at[pl.ds(0, 128)], accl.at[pl.ds(base + 6144, 128)])
  pltpu.sync_copy(msg.at[pl.ds(0, 128)], accg.at[pl.ds(base + 6144, 128)])
  pltpu.sync_copy(p_l, psl)
  pltpu.sync_copy(p_g, psg)
  plsc.subcore_barrier()

  def run_graph(w_hbm, k_hbm, d_hbm, ps, acc, full_chunks, extra_pred,
                tail_base, tail_workers):
    lw = [ps[j] for j in range(25)]
    lb = [ps[25 + j] for j in range(5)]

    def do_rows(row0, nrows):
      e0 = row0 * 128
      cps = [
          pltpu.async_copy(w_hbm.at[pl.ds(e0, nrows * 128)],
                           wbuf.at[pl.ds(0, nrows * 128)], sem_in),
          pltpu.async_copy(k_hbm.at[pl.ds(row0 * 4, nrows * 4)],
                           kbuf.at[pl.ds(0, nrows * 4)], sem_in),
      ] + [
          pltpu.async_copy(d_hbm.at[1, pl.ds(e0 + jr * 128, 128)],
                           dbuf.at[jr], sem_in)
          for jr in range(nrows)
      ]
      for cp in cps:
        cp.wait()

      def grp(bb, carry):
        for sub in range(8):
          wv = wbuf[pl.ds(bb * 128 + sub * 16, 16)]
          cols = sub * 16 + iota16
          kv0 = plsc.load_gather(
              kbuf, [jnp.full((16,), 4 * bb, jnp.int32), cols])
          kv1 = plsc.load_gather(
              kbuf, [jnp.full((16,), 4 * bb + 1, jnp.int32), cols])
          kv2 = plsc.load_gather(
              kbuf, [jnp.full((16,), 4 * bb + 2, jnp.int32), cols])
          kv3 = plsc.load_gather(
              kbuf, [jnp.full((16,), 4 * bb + 3, jnp.int32), cols])
          rows = bb * 128 + sub * 16 + iota16
          for j in range(5):
            a = (lb[j] + lw[5 * j] * wv + lw[5 * j + 1] * kv0
                 + lw[5 * j + 2] * kv1 + lw[5 * j + 3] * kv2
                 + lw[5 * j + 4] * kv3)
            plsc.store_scatter(msg, [rows, jnp.full((16,), j, jnp.int32)],
                               jnp.maximum(a, 0.0))
        return carry

      lax.fori_loop(0, nrows, grp, 0)
      hs = [
          pltpu.async_copy(msg.at[pl.ds(jr * 128, 128)],
                           acc.at[dbuf.at[jr]], sem_sc, add=True)
          for jr in range(nrows)
      ]
      for h in hs:
        h.wait()

    def chunk(i, carry):
      do_rows((w + 32 * i) * 8, 8)
      return carry

    lax.fori_loop(0, full_chunks, chunk, 0)

    @pl.when(w < tail_workers)
    def _():
      do_rows(tail_base + w, 1)

  run_graph(w_l, k_l, d_l, psl, accl,
            97 + jnp.where(w < 21, 1, 0).astype(jnp.int32), None, 25000, 0)
  run_graph(w_g, k_g, d_g, psg, accg,
            48 + jnp.where(w < 26, 1, 0).astype(jnp.int32), None, 12496, 4)

  plsc.subcore_barrier()
  pltpu.sync_copy(accl.at[pl.ds(base, _RPT)], out_l.at[c, pl.ds(base, _RPT)])
  pltpu.sync_copy(accg.at[pl.ds(base, _RPT)], out_g.at[c, pl.ds(base, _RPT)])


_sc_edges = functools.partial(
    pl.kernel,
    out_type=[
        jax.ShapeDtypeStruct((2, _NPAD, 8), jnp.float32),
        jax.ShapeDtypeStruct((2, _NPAD, 8), jnp.float32),
    ],
    mesh=plsc.VectorSubcoreMesh(core_axis_name="c", subcore_axis_name="s"),
    compiler_params=pltpu.CompilerParams(needs_layout_passes=False,
                                         use_tc_tiling_on_sc=False),
    scratch_types=[
        pltpu.VMEM_SHARED((_NPAD, 8), jnp.float32),
        pltpu.VMEM_SHARED((_NPAD, 8), jnp.float32),
        pltpu.VMEM((32, 16), jnp.float32),
        pltpu.VMEM((32, 16), jnp.float32),
        pltpu.VMEM((1024,), jnp.float32),
        pltpu.VMEM((32, 128), jnp.float32),
        pltpu.VMEM((8, 128), jnp.int32),
        pltpu.VMEM((1024, 8), jnp.float32),
        pltpu.SemaphoreType.DMA,
        pltpu.SemaphoreType.DMA,
    ],
)(_sc_body)


def _tc_body(al_ref, ag_ref, bt_ref, w1l, b1l, w2l, b2l, w1g, b1g, w2g, b2g,
             wv, bv, wo, bo, wa, ba, pp, pb, out_ref, sums, cnts, gsum):
  i = pl.program_id(0)

  @pl.when(i == 0)
  def _():
    sums[...] = jnp.zeros((1024, 128), jnp.float32)
    cnts[...] = jnp.zeros((1024, 8), jnp.float32)
    gsum[...] = jnp.zeros((1, 128), jnp.float32)

  @pl.when(i < _NT)
  def _():
    al = al_ref[0] + al_ref[1] + 1.0
    ul = jnp.maximum(
        jnp.dot(al, w1l[...], preferred_element_type=jnp.float32) + b1l[...],
        0.0)
    ml = jnp.dot(ul, w2l[...], preferred_element_type=jnp.float32) + b2l[...]
    xl = jnp.where(ml > 0, ml, 0.01 * ml)
    bt = bt_ref[0]
    seg = lax.broadcasted_iota(jnp.int32, (1024, 512), 0)
    oh = (seg == jnp.broadcast_to(bt, (1024, 512))).astype(jnp.bfloat16)
    sums[...] += jnp.dot(oh, xl.astype(jnp.bfloat16),
                         preferred_element_type=jnp.float32)
    cnts[...] += jnp.dot(oh, jnp.ones((512, 8), jnp.bfloat16),
                         preferred_element_type=jnp.float32)

    ag = ag_ref[0] + ag_ref[1] + 1.0
    ug = jnp.maximum(
        jnp.dot(ag, w1g[...], preferred_element_type=jnp.float32) + b1g[...],
        0.0)
    mg = jnp.dot(ug, w2g[...], preferred_element_type=jnp.float32) + b2g[...]
    xg = jnp.where(mg > 0, mg, 0.01 * mg)
    rid = lax.broadcasted_iota(jnp.int32, (512, 128), 0) + i * 512
    xg = jnp.where(rid < _NL, xg, 0.0)
    gsum[...] += jnp.sum(xg, axis=0, keepdims=True)

  @pl.when(i == _NT)
  def _():
    cnt = jnp.maximum(cnts[...][:, 0:1], 1.0)
    emb = sums[...] / cnt
    gm = gsum[...] * (1.0 / _NL)
    v1 = jnp.dot(gm, wv[...], preferred_element_type=jnp.float32) + bv[...]
    v2 = jnp.dot(v1, wo[...], preferred_element_type=jnp.float32) + bo[...]
    cv = jnp.dot(v2, wa[...], preferred_element_type=jnp.float32) + ba[...]
    fused = emb + cv
    out_ref[...] = jnp.dot(pp[...], fused,
                           preferred_element_type=jnp.float32) + pb[...]


def _const2(i):
  return (0, 0)


def _tc_finish(aggr_l, aggr_g, batch3d, args):
  specs = [
      pl.BlockSpec((2, 512, 8), lambda i: (0, jnp.minimum(i, _NT - 1), 0)),
      pl.BlockSpec((2, 512, 8), lambda i: (0, jnp.minimum(i, _NT - 1), 0)),
      pl.BlockSpec((1, 1, 512), lambda i: (jnp.minimum(i, _NT - 1), 0, 0)),
  ] + [pl.BlockSpec(a.shape, _const2) for a in args]
  return pl.pallas_call(
      _tc_body,
      grid=(_NT + 1,),
      in_specs=specs,
      out_specs=pl.BlockSpec((1024, 128), _const2),
      out_shape=jax.ShapeDtypeStruct((1024, 128), jnp.float32),
      scratch_shapes=[
          pltpu.VMEM((1024, 128), jnp.float32),
          pltpu.VMEM((1024, 8), jnp.float32),
          pltpu.VMEM((1, 128), jnp.float32),
      ],
  )(aggr_l, aggr_g, batch3d, *args)


def kernel(local_edge_index, local_weight, local_kmer, local_batch,
           global_edge_index, global_weight, global_kmer,
           lin_local_w, lin_local_b, mlp_local_w1, mlp_local_b1,
           mlp_local_w2, mlp_local_b2, lin_global_w, lin_global_b,
           mlp_global_w1, mlp_global_b1, mlp_global_w2, mlp_global_b2,
           in_proj_w, in_proj_b, out_proj_w, out_proj_b,
           attn_proj_w, attn_proj_b, project_w, project_b):
  f32 = jnp.float32
  dl = local_edge_index.astype(jnp.int32)
  dg = global_edge_index.astype(jnp.int32)
  kl = local_kmer.reshape(_ROWS_L, 128, 4).transpose(0, 2, 1).reshape(
      _ROWS_L * 4, 128)
  kg = global_kmer.T.reshape(4, _ROWS_G, 128).transpose(1, 0, 2).reshape(
      _ROWS_G * 4, 128)

  def mk_params(lw, lb):
    m = jnp.concatenate([lw.reshape(25), 1.0 + lb, jnp.zeros((2,), f32)])
    return jnp.tile(m[:, None], (1, 16)).astype(f32)

  aggr_l, aggr_g = _sc_edges(
      local_weight, kl, dl, mk_params(lin_local_w, lin_local_b),
      global_weight, kg, dg, mk_params(lin_global_w, lin_global_b),
      jnp.zeros((1024, 8), f32))

  batch3d = jnp.pad(local_batch.astype(jnp.int32), (0, _NPAD - _NL),
                    constant_values=1023).reshape(_NT, 1, 512)

  def w1pad(w1):
    return jnp.zeros((8, 128), f32).at[:5].set(w1.T)

  args = [
      w1pad(mlp_local_w1), mlp_local_b1[None], mlp_local_w2.T,
      mlp_local_b2[None],
      w1pad(mlp_global_w1), mlp_global_b1[None], mlp_global_w2.T,
      mlp_global_b2[None],
      in_proj_w[256:384].T, in_proj_b[256:384][None],
      out_proj_w.T, out_proj_b[None],
      attn_proj_w.T, attn_proj_b[None],
      jnp.zeros((1024, 1024), f32).at[:, :_NSEG].set(project_w),
      jnp.broadcast_to(project_b[:, None], (1024, 128)),
  ]
  out1 = _tc_finish(aggr_l, aggr_g, batch3d, args)
  return (out1.reshape(1, _SEQ, 128), jnp.ones((_NSEG, 1, 1), f32))


# speedup vs baseline: 9.5530x; 1.0739x over previous
import functools

import jax
import jax.numpy as jnp
from jax import lax
from jax.experimental import pallas as pl
from jax.experimental.pallas import tpu as pltpu
from jax.experimental.pallas import tpu_sc as plsc

_NL = 100000
_EL = 3200000
_EG = 1600000
_NSEG = 1022
_SEQ = 1024
_NPAD = 100352
_NT = 98
_ROWS_L = _EL // 128
_ROWS_G = _EG // 128
_RPT = _NPAD // 16


def _sc_body(w_l, k_l, d_l, p_l, w_g, k_g, d_g, p_g, zz,
             out_l, out_g,
             accl, accg, psl, psg, wbuf, kbuf, dbuf, msg, sem_in, sem_sc):
  c = lax.axis_index("c")
  s = lax.axis_index("s")
  w = s * 2 + c
  iota16 = lax.iota(jnp.int32, 16)

  pltpu.sync_copy(zz, msg)
  base = s * _RPT
  for t in range(6):
    pltpu.sync_copy(msg, accl.at[pl.ds(base + t * 1024, 1024)])
    pltpu.sync_copy(msg, accg.at[pl.ds(base + t * 1024, 1024)])
  pltpu.sync_copy(msg.at[pl.ds(0, 128)], accl.at[pl.ds(base + 6144, 128)])
  pltpu.sync_copy(msg.at[pl.ds(0, 128)], accg.at[pl.ds(base + 6144, 128)])
  pltpu.sync_copy(p_l, psl)
  pltpu.sync_copy(p_g, psg)
  plsc.subcore_barrier()

  def run_graph(w_hbm, k_hbm, d_hbm, ps, acc, full_chunks, extra_pred,
                tail_base, tail_workers):
    lw = [ps[j] for j in range(25)]
    lb = [ps[25 + j] for j in range(5)]

    def do_rows(row0, nrows):
      e0 = row0 * 128
      cps = [
          pltpu.async_copy(w_hbm.at[pl.ds(e0, nrows * 128)],
                           wbuf.at[pl.ds(0, nrows * 128)], sem_in),
          pltpu.async_copy(k_hbm.at[pl.ds(row0 * 4, nrows * 4)],
                           kbuf.at[pl.ds(0, nrows * 4)], sem_in),
      ] + [
          pltpu.async_copy(d_hbm.at[1, pl.ds(e0 + jr * 128, 128)],
                           dbuf.at[jr], sem_in)
          for jr in range(nrows)
      ]
      for cp in cps:
        cp.wait()

      def grp(bb, carry):
        for sub in range(8):
          wv = wbuf[pl.ds(bb * 128 + sub * 16, 16)]
          cols = sub * 16 + iota16
          kv0 = plsc.load_gather(
              kbuf, [jnp.full((16,), 4 * bb, jnp.int32), cols])
          kv1 = plsc.load_gather(
              kbuf, [jnp.full((16,), 4 * bb + 1, jnp.int32), cols])
          kv2 = plsc.load_gather(
              kbuf, [jnp.full((16,), 4 * bb + 2, jnp.int32), cols])
          kv3 = plsc.load_gather(
              kbuf, [jnp.full((16,), 4 * bb + 3, jnp.int32), cols])
          rows = bb * 128 + sub * 16 + iota16
          for j in range(5):
            a = (lb[j] + lw[5 * j] * wv + lw[5 * j + 1] * kv0
                 + lw[5 * j + 2] * kv1 + lw[5 * j + 3] * kv2
                 + lw[5 * j + 4] * kv3)
            plsc.store_scatter(msg, [rows, jnp.full((16,), j, jnp.int32)],
                               jnp.maximum(a, 0.0))
        return carry

      lax.fori_loop(0, nrows, grp, 0)
      hs = [
          pltpu.async_copy(msg.at[pl.ds(jr * 128, 128)],
                           acc.at[dbuf.at[jr]], sem_sc, add=True)
          for jr in range(nrows)
      ]
      for h in hs:
        h.wait()

    def chunk(i, carry):
      do_rows((w + 32 * i) * 8, 8)
      return carry

    lax.fori_loop(0, full_chunks, chunk, 0)

    @pl.when(w < tail_workers)
    def _():
      do_rows(tail_base + w, 1)

  run_graph(w_l, k_l, d_l, psl, accl,
            97 + jnp.where(w < 21, 1, 0).astype(jnp.int32), None, 25000, 0)
  run_graph(w_g, k_g, d_g, psg, accg,
            48 + jnp.where(w < 26, 1, 0).astype(jnp.int32), None, 12496, 4)

  plsc.subcore_barrier()
  pltpu.sync_copy(accl.at[pl.ds(base, _RPT)], out_l.at[c, pl.ds(base, _RPT)])
  pltpu.sync_copy(accg.at[pl.ds(base, _RPT)], out_g.at[c, pl.ds(base, _RPT)])


_sc_edges = functools.partial(
    pl.kernel,
    out_type=[
        jax.ShapeDtypeStruct((2, _NPAD, 8), jnp.float32),
        jax.ShapeDtypeStruct((2, _NPAD, 8), jnp.float32),
    ],
    mesh=plsc.VectorSubcoreMesh(core_axis_name="c", subcore_axis_name="s"),
    compiler_params=pltpu.CompilerParams(needs_layout_passes=False,
                                         use_tc_tiling_on_sc=False),
    scratch_types=[
        pltpu.VMEM_SHARED((_NPAD, 8), jnp.float32),
        pltpu.VMEM_SHARED((_NPAD, 8), jnp.float32),
        pltpu.VMEM((32, 16), jnp.float32),
        pltpu.VMEM((32, 16), jnp.float32),
        pltpu.VMEM((1024,), jnp.float32),
        pltpu.VMEM((32, 128), jnp.float32),
        pltpu.VMEM((8, 128), jnp.int32),
        pltpu.VMEM((1024, 8), jnp.float32),
        pltpu.SemaphoreType.DMA,
        pltpu.SemaphoreType.DMA,
    ],
)(_sc_body)


def _tc_body(al_ref, ag_ref, bt_ref, w1l, b1l, w2l, b2l, w1g, b1g, w2g, b2g,
             wv, bv, wo, bo, wa, ba, pp, pb, out_ref, sums, cnts, gsum):
  i = pl.program_id(0)

  @pl.when(i == 0)
  def _():
    sums[...] = jnp.zeros((1024, 128), jnp.float32)
    cnts[...] = jnp.zeros((1024, 8), jnp.float32)
    gsum[...] = jnp.zeros((1, 128), jnp.float32)

  @pl.when(i < _NT)
  def _():
    al = al_ref[0] + al_ref[1] + 1.0
    ul = jnp.maximum(
        jnp.dot(al, w1l[...], preferred_element_type=jnp.float32) + b1l[...],
        0.0)
    ml = jnp.dot(ul, w2l[...], preferred_element_type=jnp.float32) + b2l[...]
    xl = jnp.where(ml > 0, ml, 0.01 * ml)
    bt = bt_ref[0]
    seg = lax.broadcasted_iota(jnp.int32, (1024, 1024), 0)
    oh = (seg == jnp.broadcast_to(bt, (1024, 1024))).astype(jnp.bfloat16)
    sums[...] += jnp.dot(oh, xl.astype(jnp.bfloat16),
                         preferred_element_type=jnp.float32)
    cnts[...] += jnp.dot(oh, jnp.ones((1024, 8), jnp.bfloat16),
                         preferred_element_type=jnp.float32)

    ag = ag_ref[0] + ag_ref[1] + 1.0
    ug = jnp.maximum(
        jnp.dot(ag, w1g[...], preferred_element_type=jnp.float32) + b1g[...],
        0.0)
    mg = jnp.dot(ug, w2g[...], preferred_element_type=jnp.float32) + b2g[...]
    xg = jnp.where(mg > 0, mg, 0.01 * mg)
    rid = lax.broadcasted_iota(jnp.int32, (1024, 128), 0) + i * 1024
    xg = jnp.where(rid < _NL, xg, 0.0)
    gsum[...] += jnp.sum(xg, axis=0, keepdims=True)

  @pl.when(i == _NT)
  def _():
    cnt = jnp.maximum(cnts[...][:, 0:1], 1.0)
    emb = sums[...] / cnt
    gm = gsum[...] * (1.0 / _NL)
    v1 = jnp.dot(gm, wv[...], preferred_element_type=jnp.float32) + bv[...]
    v2 = jnp.dot(v1, wo[...], preferred_element_type=jnp.float32) + bo[...]
    cv = jnp.dot(v2, wa[...], preferred_element_type=jnp.float32) + ba[...]
    fused = emb + cv
    out_ref[...] = jnp.dot(pp[...], fused,
                           preferred_element_type=jnp.float32) + pb[...]


def _const2(i):
  return (0, 0)


def _tc_finish(aggr_l, aggr_g, batch3d, args):
  specs = [
      pl.BlockSpec((2, 1024, 8), lambda i: (0, jnp.minimum(i, _NT - 1), 0)),
      pl.BlockSpec((2, 1024, 8), lambda i: (0, jnp.minimum(i, _NT - 1), 0)),
      pl.BlockSpec((1, 1, 1024), lambda i: (jnp.minimum(i, _NT - 1), 0, 0)),
  ] + [pl.BlockSpec(a.shape, _const2) for a in args]
  return pl.pallas_call(
      _tc_body,
      grid=(_NT + 1,),
      in_specs=specs,
      out_specs=pl.BlockSpec((1024, 128), _const2),
      out_shape=jax.ShapeDtypeStruct((1024, 128), jnp.float32),
      scratch_shapes=[
          pltpu.VMEM((1024, 128), jnp.float32),
          pltpu.VMEM((1024, 8), jnp.float32),
          pltpu.VMEM((1, 128), jnp.float32),
      ],
  )(aggr_l, aggr_g, batch3d, *args)


def kernel(local_edge_index, local_weight, local_kmer, local_batch,
           global_edge_index, global_weight, global_kmer,
           lin_local_w, lin_local_b, mlp_local_w1, mlp_local_b1,
           mlp_local_w2, mlp_local_b2, lin_global_w, lin_global_b,
           mlp_global_w1, mlp_global_b1, mlp_global_w2, mlp_global_b2,
           in_proj_w, in_proj_b, out_proj_w, out_proj_b,
           attn_proj_w, attn_proj_b, project_w, project_b):
  f32 = jnp.float32
  dl = local_edge_index.astype(jnp.int32)
  dg = global_edge_index.astype(jnp.int32)
  kl = local_kmer.reshape(_ROWS_L, 128, 4).transpose(0, 2, 1).reshape(
      _ROWS_L * 4, 128)
  kg = global_kmer.T.reshape(4, _ROWS_G, 128).transpose(1, 0, 2).reshape(
      _ROWS_G * 4, 128)

  def mk_params(lw, lb):
    m = jnp.concatenate([lw.reshape(25), 1.0 + lb, jnp.zeros((2,), f32)])
    return jnp.tile(m[:, None], (1, 16)).astype(f32)

  aggr_l, aggr_g = _sc_edges(
      local_weight, kl, dl, mk_params(lin_local_w, lin_local_b),
      global_weight, kg, dg, mk_params(lin_global_w, lin_global_b),
      jnp.zeros((1024, 8), f32))

  batch3d = jnp.pad(local_batch.astype(jnp.int32), (0, _NPAD - _NL),
                    constant_values=1023).reshape(_NT, 1, 1024)

  def w1pad(w1):
    return jnp.zeros((8, 128), f32).at[:5].set(w1.T)

  args = [
      w1pad(mlp_local_w1), mlp_local_b1[None], mlp_local_w2.T,
      mlp_local_b2[None],
      w1pad(mlp_global_w1), mlp_global_b1[None], mlp_global_w2.T,
      mlp_global_b2[None],
      in_proj_w[256:384].T, in_proj_b[256:384][None],
      out_proj_w.T, out_proj_b[None],
      attn_proj_w.T, attn_proj_b[None],
      jnp.zeros((1024, 1024), f32).at[:, :_NSEG].set(project_w),
      jnp.broadcast_to(project_b[:, None], (1024, 128)),
  ]
  out1 = _tc_finish(aggr_l, aggr_g, batch3d, args)
  return (out1.reshape(1, _SEQ, 128), jnp.ones((_NSEG, 1, 1), f32))
